# Initial kernel scaffold; baseline (speedup 1.0000x reference)
import jax
import jax.numpy as jnp
from jax.experimental import pallas as pl


def kernel(x, edge_index, nodes, parents, first_message, W_enc, b_enc, W_ns, b_ns, W_nm, b_nm, W_dec, b_dec):
    raise NotImplementedError("write your pallas kernel here")


# speedup vs baseline: 6.8325x; 6.8325x over previous
import functools

import jax
import jax.numpy as jnp
from jax import lax
from jax.experimental import pallas as pl
from jax.experimental.pallas import tpu as pltpu

N = 64
S = 64
T = 320
IN_F = 128
HID = 64
MSG = 32
OUT_F = 10
CH = 32


def _body(x_ref, nodesT_ref, parentsT_ref, first_ref,
          W_encT_ref, b_enc_ref, W_nsT_ref, b_ns_ref,
          W_nmT_ref, b_nm_ref, W_decT_ref, b_dec_ref,
          out_ref, pred_ref, store_ref):
    f32 = jnp.float32
    enc = jnp.dot(x_ref[...], W_encT_ref[...],
                  preferred_element_type=f32) + b_enc_ref[...]
    pred_ref[...] = jnp.broadcast_to(enc[:, None, :], (N, S, HID))
    store_ref[...] = jnp.zeros((T, S, MSG), f32)

    def step(t, carry):
        nd = nodesT_ref[pl.ds(t, 1), :]
        pa = parentsT_ref[pl.ds(t, 1), :]
        pac = jnp.maximum(pa, 0).reshape(1, S, 1)

        def chunk(c, acc):
            base = c * CH
            io = lax.broadcasted_iota(jnp.int32, (CH, S, 1), 0) + base
            blk = store_ref[pl.ds(base, CH)]
            return acc + jnp.sum(jnp.where(io == pac, blk, 0.0), axis=0)

        nchunks = t // CH + 1
        prev = lax.fori_loop(0, nchunks, chunk, jnp.zeros((S, MSG), f32))
        msg = jnp.where(pa.reshape(S, 1) < 0, first_ref[0, :][None, :], prev)

        ndm = nd.reshape(1, S, 1)
        io_n = lax.broadcasted_iota(jnp.int32, (N, S, 1), 0)
        mask3 = io_n == ndm
        pred_all = pred_ref[...]
        feat = jnp.sum(jnp.where(mask3, pred_all, 0.0), axis=0)

        cat = jnp.concatenate([feat, msg], axis=1)
        newstate = jax.nn.relu(
            jnp.dot(cat, W_nsT_ref[...], preferred_element_type=f32)
            + b_ns_ref[...])
        cat2 = jnp.concatenate([newstate, msg], axis=1)
        newmsg = (jnp.dot(cat2, W_nmT_ref[...], preferred_element_type=f32)
                  + b_nm_ref[...])

        pred_ref[...] = jnp.where(mask3, newstate[None, :, :], pred_all)
        store_ref[pl.ds(t, 1)] = newmsg.reshape(1, S, MSG)
        return carry

    lax.fori_loop(0, T, step, 0, unroll=False)

    io_n = lax.broadcasted_iota(jnp.int32, (N, S, 1), 0)
    io_s = lax.broadcasted_iota(jnp.int32, (N, S, 1), 1)
    final = jnp.sum(jnp.where(io_n == io_s, pred_ref[...], 0.0), axis=0)
    logits = (jnp.dot(final, W_decT_ref[...], preferred_element_type=f32)
              + b_dec_ref[...])
    mx = jnp.max(logits, axis=1, keepdims=True)
    sh = logits - mx
    lse = jnp.log(jnp.sum(jnp.exp(sh), axis=1, keepdims=True))
    out_ref[...] = sh - lse


@functools.partial(jax.jit, static_argnames=())
def kernel(x, edge_index, nodes, parents, first_message,
           W_enc, b_enc, W_ns, b_ns, W_nm, b_nm, W_dec, b_dec):
    del edge_index
    f32 = jnp.float32
    call = pl.pallas_call(
        _body,
        out_shape=jax.ShapeDtypeStruct((S, OUT_F), f32),
        scratch_shapes=[
            pltpu.VMEM((N, S, HID), f32),
            pltpu.VMEM((T, S, MSG), f32),
        ],
    )
    return call(
        x,
        nodes.astype(jnp.int32).T,
        parents.astype(jnp.int32).T,
        first_message,
        W_enc.T, b_enc.reshape(1, HID),
        W_ns.T, b_ns.reshape(1, HID),
        W_nm.T, b_nm.reshape(1, MSG),
        W_dec.T, b_dec.reshape(1, OUT_F),
    )


# speedup vs baseline: 19.9444x; 2.9191x over previous
import functools

import jax
import jax.numpy as jnp
from jax import lax
from jax.experimental import pallas as pl
from jax.experimental.pallas import tpu as pltpu
from jax.experimental.pallas import tpu_sc as plsc

N = 64
S = 64
T = 320
IN_F = 128
HID = 64
MSG = 32
OUT_F = 10
NC = 2
NS = 16
NW = NC * NS
SIMS = S // NW
L = 16


def _encode_body(x_ref, w_ref, b_ref, out_ref):
    out_ref[...] = (jnp.dot(x_ref[...], w_ref[...],
                            preferred_element_type=jnp.float32) + b_ref[...])


def _decode_body(f_ref, w_ref, b_ref, out_ref):
    logits = (jnp.dot(f_ref[...], w_ref[...],
                      preferred_element_type=jnp.float32) + b_ref[...])
    mx = jnp.max(logits, axis=1, keepdims=True)
    sh = logits - mx
    out_ref[...] = sh - jnp.log(jnp.sum(jnp.exp(sh), axis=1, keepdims=True))


def _sim_body(enc_hbm, sched_hbm, first_hbm,
              wns_hbm, bns_hbm, wnm_hbm, bnm_hbm, out_hbm,
              pred0, pred1, msgs0, msgs1, sch0, sch1,
              wns, wnm, bns, bnm, fst, fin):
    wid = lax.axis_index("s") * NC + lax.axis_index("c")
    s0 = wid * SIMS

    pltpu.sync_copy(enc_hbm, pred0)
    pltpu.sync_copy(enc_hbm, pred1)
    pltpu.sync_copy(sched_hbm.at[pl.ds(s0 * T * L, T * L)], sch0)
    pltpu.sync_copy(sched_hbm.at[pl.ds((s0 + 1) * T * L, T * L)], sch1)
    pltpu.sync_copy(wns_hbm, wns)
    pltpu.sync_copy(wnm_hbm, wnm)
    pltpu.sync_copy(bns_hbm, bns)
    pltpu.sync_copy(bnm_hbm, bnm)
    pltpu.sync_copy(first_hbm, fst)

    fstv = [fst[pl.ds(0, L)], fst[pl.ds(L, L)]]

    def step(t, carry):
        sv0 = sch0[pl.ds(t * L, L)]
        sv1 = sch1[pl.ds(t * L, L)]
        nd0 = sv0[0]
        pa0 = sv0[1]
        nd1 = sv1[0]
        pa1 = sv1[1]
        uf0 = pa0 < 0
        uf1 = pa1 < 0
        p0 = jnp.maximum(pa0, 0)
        p1 = jnp.maximum(pa1, 0)

        a0 = [bns[pl.ds(hb * L, L)] for hb in range(4)]
        a1 = [bns[pl.ds(hb * L, L)] for hb in range(4)]
        m0 = [bnm[pl.ds(mb * L, L)] for mb in range(2)]
        m1 = [bnm[pl.ds(mb * L, L)] for mb in range(2)]

        fv0 = [pred0[pl.ds(nd0 * HID + hb * L, L)] for hb in range(4)]
        fv1 = [pred1[pl.ds(nd1 * HID + hb * L, L)] for hb in range(4)]
        mv0 = [msgs0[pl.ds(p0 * MSG + mb * L, L)] for mb in range(2)]
        mv1 = [msgs1[pl.ds(p1 * MSG + mb * L, L)] for mb in range(2)]

        for k in range(HID):
            f0 = fv0[k // L][k % L]
            f1 = fv1[k // L][k % L]
            for hb in range(4):
                w = wns[pl.ds(k * HID + hb * L, L)]
                a0[hb] = a0[hb] + w * f0
                a1[hb] = a1[hb] + w * f1

        for k in range(MSG):
            v0 = jnp.where(uf0, fstv[k // L][k % L], mv0[k // L][k % L])
            v1 = jnp.where(uf1, fstv[k // L][k % L], mv1[k // L][k % L])
            kk = HID + k
            for hb in range(4):
                w = wns[pl.ds(kk * HID + hb * L, L)]
                a0[hb] = a0[hb] + w * v0
                a1[hb] = a1[hb] + w * v1
            for mb in range(2):
                w = wnm[pl.ds(kk * MSG + mb * L, L)]
                m0[mb] = m0[mb] + w * v0
                m1[mb] = m1[mb] + w * v1

        ns0 = [jnp.maximum(a0[hb], 0.0) for hb in range(4)]
        ns1 = [jnp.maximum(a1[hb], 0.0) for hb in range(4)]
        for hb in range(4):
            pred0[pl.ds(nd0 * HID + hb * L, L)] = ns0[hb]
            pred1[pl.ds(nd1 * HID + hb * L, L)] = ns1[hb]

        for k in range(HID):
            n0 = ns0[k // L][k % L]
            n1 = ns1[k // L][k % L]
            for mb in range(2):
                w = wnm[pl.ds(k * MSG + mb * L, L)]
                m0[mb] = m0[mb] + w * n0
                m1[mb] = m1[mb] + w * n1

        for mb in range(2):
            msgs0[pl.ds(t * MSG + mb * L, L)] = m0[mb]
            msgs1[pl.ds(t * MSG + mb * L, L)] = m1[mb]
        return carry

    lax.fori_loop(0, T, step, 0)

    for hb in range(4):
        fin[pl.ds(hb * L, L)] = pred0[pl.ds(s0 * HID + hb * L, L)]
        fin[pl.ds(HID + hb * L, L)] = pred1[pl.ds((s0 + 1) * HID + hb * L, L)]
    pltpu.sync_copy(fin, out_hbm.at[pl.ds(s0 * HID, SIMS * HID)])


def kernel(x, edge_index, nodes, parents, first_message,
           W_enc, b_enc, W_ns, b_ns, W_nm, b_nm, W_dec, b_dec):
    del edge_index
    f32 = jnp.float32

    enc = pl.pallas_call(
        _encode_body,
        out_shape=jax.ShapeDtypeStruct((N, HID), f32),
    )(x, W_enc.T, b_enc.reshape(1, HID))

    sim = pl.kernel(
        _sim_body,
        out_type=jax.ShapeDtypeStruct((S * HID,), f32),
        mesh=plsc.VectorSubcoreMesh(core_axis_name="c", subcore_axis_name="s"),
        scratch_types=[
            pltpu.VMEM((N * HID,), f32),
            pltpu.VMEM((N * HID,), f32),
            pltpu.VMEM((T * MSG,), f32),
            pltpu.VMEM((T * MSG,), f32),
            pltpu.VMEM((T * L,), jnp.int32),
            pltpu.VMEM((T * L,), jnp.int32),
            pltpu.VMEM(((HID + MSG) * HID,), f32),
            pltpu.VMEM(((HID + MSG) * MSG,), f32),
            pltpu.VMEM((HID,), f32),
            pltpu.VMEM((MSG,), f32),
            pltpu.VMEM((MSG,), f32),
            pltpu.VMEM((SIMS * HID,), f32),
        ],
    )
    sched = jnp.stack([nodes.astype(jnp.int32), parents.astype(jnp.int32)],
                      axis=-1)
    sched = jnp.pad(sched, ((0, 0), (0, 0), (0, L - 2)))
    final = sim(enc.reshape(N * HID),
                sched.reshape(S * T * L),
                first_message.reshape(MSG),
                W_ns.T.reshape((HID + MSG) * HID),
                b_ns, W_nm.T.reshape((HID + MSG) * MSG), b_nm)

    return pl.pallas_call(
        _decode_body,
        out_shape=jax.ShapeDtypeStruct((S, OUT_F), f32),
    )(final.reshape(S, HID), W_dec.T, b_dec.reshape(1, OUT_F))


# speedup vs baseline: 20.0274x; 1.0042x over previous
import functools

import jax
import jax.numpy as jnp
from jax import lax
from jax.experimental import pallas as pl
from jax.experimental.pallas import tpu as pltpu
from jax.experimental.pallas import tpu_sc as plsc

N = 64
S = 64
T = 320
IN_F = 128
HID = 64
MSG = 32
OUT_F = 10
NC = 2
NS = 16
NW = NC * NS
SIMS = S // NW
L = 16


def _encode_body(x_ref, w_ref, b_ref, out_ref):
    out_ref[...] = (jnp.dot(x_ref[...], w_ref[...],
                            preferred_element_type=jnp.float32) + b_ref[...])


def _decode_body(f_ref, w_ref, b_ref, out_ref):
    logits = (jnp.dot(f_ref[...], w_ref[...],
                      preferred_element_type=jnp.float32) + b_ref[...])
    mx = jnp.max(logits, axis=1, keepdims=True)
    sh = logits - mx
    out_ref[...] = sh - jnp.log(jnp.sum(jnp.exp(sh), axis=1, keepdims=True))


def _sim_body(enc_hbm, sched_hbm, first_hbm,
              wns_hbm, bns_hbm, wnm_hbm, bnm_hbm, out_hbm,
              pred0, pred1, msgs0, msgs1, sch0, sch1,
              wns, wnm, bns, bnm, fst, fin):
    wid = lax.axis_index("s") * NC + lax.axis_index("c")
    s0 = wid * SIMS

    pltpu.sync_copy(enc_hbm, pred0)
    pltpu.sync_copy(enc_hbm, pred1)
    pltpu.sync_copy(sched_hbm.at[pl.ds(s0 * T * L, T * L)], sch0)
    pltpu.sync_copy(sched_hbm.at[pl.ds((s0 + 1) * T * L, T * L)], sch1)
    pltpu.sync_copy(wns_hbm, wns)
    pltpu.sync_copy(wnm_hbm, wnm)
    pltpu.sync_copy(bns_hbm, bns)
    pltpu.sync_copy(bnm_hbm, bnm)
    pltpu.sync_copy(first_hbm, fst)

    fstv = [fst[pl.ds(0, L)], fst[pl.ds(L, L)]]

    def step(t, carry):
        sv0 = sch0[pl.ds(t * L, L)]
        sv1 = sch1[pl.ds(t * L, L)]
        nd0 = sv0[0]
        pa0 = sv0[1]
        nd1 = sv1[0]
        pa1 = sv1[1]
        uf0 = pa0 < 0
        uf1 = pa1 < 0
        p0 = jnp.maximum(pa0, 0)
        p1 = jnp.maximum(pa1, 0)

        a0 = [bns[pl.ds(hb * L, L)] for hb in range(4)]
        a1 = [bns[pl.ds(hb * L, L)] for hb in range(4)]
        m0 = [bnm[pl.ds(mb * L, L)] for mb in range(2)]
        m1 = [bnm[pl.ds(mb * L, L)] for mb in range(2)]

        fv0 = [pred0[pl.ds(nd0 * HID + hb * L, L)] for hb in range(4)]
        fv1 = [pred1[pl.ds(nd1 * HID + hb * L, L)] for hb in range(4)]
        mv0 = [jnp.where(uf0, fstv[mb], msgs0[pl.ds(p0 * MSG + mb * L, L)])
               for mb in range(2)]
        mv1 = [jnp.where(uf1, fstv[mb], msgs1[pl.ds(p1 * MSG + mb * L, L)])
               for mb in range(2)]

        for k in range(HID):
            f0 = jnp.broadcast_to(fv0[k // L][k % L], (L,))
            f1 = jnp.broadcast_to(fv1[k // L][k % L], (L,))
            for hb in range(4):
                w = wns[pl.ds(k * HID + hb * L, L)]
                a0[hb] = a0[hb] + w * f0
                a1[hb] = a1[hb] + w * f1

        for k in range(MSG):
            v0 = jnp.broadcast_to(mv0[k // L][k % L], (L,))
            v1 = jnp.broadcast_to(mv1[k // L][k % L], (L,))
            kk = HID + k
            for hb in range(4):
                w = wns[pl.ds(kk * HID + hb * L, L)]
                a0[hb] = a0[hb] + w * v0
                a1[hb] = a1[hb] + w * v1
            for mb in range(2):
                w = wnm[pl.ds(kk * MSG + mb * L, L)]
                m0[mb] = m0[mb] + w * v0
                m1[mb] = m1[mb] + w * v1

        ns0 = [jnp.maximum(a0[hb], 0.0) for hb in range(4)]
        ns1 = [jnp.maximum(a1[hb], 0.0) for hb in range(4)]
        for hb in range(4):
            pred0[pl.ds(nd0 * HID + hb * L, L)] = ns0[hb]
            pred1[pl.ds(nd1 * HID + hb * L, L)] = ns1[hb]

        for k in range(HID):
            n0 = jnp.broadcast_to(ns0[k // L][k % L], (L,))
            n1 = jnp.broadcast_to(ns1[k // L][k % L], (L,))
            for mb in range(2):
                w = wnm[pl.ds(k * MSG + mb * L, L)]
                m0[mb] = m0[mb] + w * n0
                m1[mb] = m1[mb] + w * n1

        for mb in range(2):
            msgs0[pl.ds(t * MSG + mb * L, L)] = m0[mb]
            msgs1[pl.ds(t * MSG + mb * L, L)] = m1[mb]
        return carry

    lax.fori_loop(0, T, step, 0)

    for hb in range(4):
        fin[pl.ds(hb * L, L)] = pred0[pl.ds(s0 * HID + hb * L, L)]
        fin[pl.ds(HID + hb * L, L)] = pred1[pl.ds((s0 + 1) * HID + hb * L, L)]
    pltpu.sync_copy(fin, out_hbm.at[pl.ds(s0 * HID, SIMS * HID)])


def kernel(x, edge_index, nodes, parents, first_message,
           W_enc, b_enc, W_ns, b_ns, W_nm, b_nm, W_dec, b_dec):
    del edge_index
    f32 = jnp.float32

    enc = pl.pallas_call(
        _encode_body,
        out_shape=jax.ShapeDtypeStruct((N, HID), f32),
    )(x, W_enc.T, b_enc.reshape(1, HID))

    sim = pl.kernel(
        _sim_body,
        out_type=jax.ShapeDtypeStruct((S * HID,), f32),
        mesh=plsc.VectorSubcoreMesh(core_axis_name="c", subcore_axis_name="s"),
        scratch_types=[
            pltpu.VMEM((N * HID,), f32),
            pltpu.VMEM((N * HID,), f32),
            pltpu.VMEM((T * MSG,), f32),
            pltpu.VMEM((T * MSG,), f32),
            pltpu.VMEM((T * L,), jnp.int32),
            pltpu.VMEM((T * L,), jnp.int32),
            pltpu.VMEM(((HID + MSG) * HID,), f32),
            pltpu.VMEM(((HID + MSG) * MSG,), f32),
            pltpu.VMEM((HID,), f32),
            pltpu.VMEM((MSG,), f32),
            pltpu.VMEM((MSG,), f32),
            pltpu.VMEM((SIMS * HID,), f32),
        ],
    )
    sched = jnp.stack([nodes.astype(jnp.int32), parents.astype(jnp.int32)],
                      axis=-1)
    sched = jnp.pad(sched, ((0, 0), (0, 0), (0, L - 2)))
    final = sim(enc.reshape(N * HID),
                sched.reshape(S * T * L),
                first_message.reshape(MSG),
                W_ns.T.reshape((HID + MSG) * HID),
                b_ns, W_nm.T.reshape((HID + MSG) * MSG), b_nm)

    return pl.pallas_call(
        _decode_body,
        out_shape=jax.ShapeDtypeStruct((S, OUT_F), f32),
    )(final.reshape(S, HID), W_dec.T, b_dec.reshape(1, OUT_F))


# speedup vs baseline: 20.0583x; 1.0015x over previous
import functools

import jax
import jax.numpy as jnp
from jax import lax
from jax.experimental import pallas as pl
from jax.experimental.pallas import tpu as pltpu
from jax.experimental.pallas import tpu_sc as plsc

N = 64
S = 64
T = 320
IN_F = 128
HID = 64
MSG = 32
OUT_F = 10
NC = 2
NS = 16
NW = NC * NS
SIMS = S // NW
L = 16


def _encode_body(x_ref, w_ref, b_ref, out_ref):
    out_ref[...] = (jnp.dot(x_ref[...], w_ref[...],
                            preferred_element_type=jnp.float32) + b_ref[...])


def _decode_body(f_ref, w_ref, b_ref, out_ref):
    logits = (jnp.dot(f_ref[...], w_ref[...],
                      preferred_element_type=jnp.float32) + b_ref[...])
    mx = jnp.max(logits, axis=1, keepdims=True)
    sh = logits - mx
    out_ref[...] = sh - jnp.log(jnp.sum(jnp.exp(sh), axis=1, keepdims=True))


def _sim_body(enc_hbm, sched_hbm, first_hbm,
              wns_hbm, bns_hbm, wnm_hbm, bnm_hbm, out_hbm,
              pred0, pred1, msgs0, msgs1, sch0, sch1,
              wns, wnm, bns, bnm, fst, fin):
    wid = lax.axis_index("s") * NC + lax.axis_index("c")
    s0 = wid * SIMS

    pltpu.sync_copy(enc_hbm, pred0)
    pltpu.sync_copy(enc_hbm, pred1)
    pltpu.sync_copy(sched_hbm.at[pl.ds(s0 * T * L, T * L)], sch0)
    pltpu.sync_copy(sched_hbm.at[pl.ds((s0 + 1) * T * L, T * L)], sch1)
    pltpu.sync_copy(wns_hbm, wns)
    pltpu.sync_copy(wnm_hbm, wnm)
    pltpu.sync_copy(bns_hbm, bns)
    pltpu.sync_copy(bnm_hbm, bnm)
    pltpu.sync_copy(first_hbm, fst)

    fstv = [fst[pl.ds(0, L)], fst[pl.ds(L, L)]]

    def step(t, carry):
        sv0 = sch0[pl.ds(t * L, L)]
        sv1 = sch1[pl.ds(t * L, L)]
        nd0 = sv0[0]
        pa0 = sv0[1]
        nd1 = sv1[0]
        pa1 = sv1[1]
        uf0 = pa0 < 0
        uf1 = pa1 < 0
        p0 = jnp.maximum(pa0, 0)
        p1 = jnp.maximum(pa1, 0)

        zero = jnp.zeros((L,), jnp.float32)
        a0 = [[bns[pl.ds(hb * L, L)] for hb in range(4)], [zero] * 4]
        a1 = [[bns[pl.ds(hb * L, L)] for hb in range(4)], [zero] * 4]
        m0 = [[bnm[pl.ds(mb * L, L)] for mb in range(2)], [zero] * 2]
        m1 = [[bnm[pl.ds(mb * L, L)] for mb in range(2)], [zero] * 2]

        iv0 = [pred0[pl.ds(nd0 * HID + hb * L, L)] for hb in range(4)]
        iv1 = [pred1[pl.ds(nd1 * HID + hb * L, L)] for hb in range(4)]
        iv0 += [jnp.where(uf0, fstv[mb], msgs0[pl.ds(p0 * MSG + mb * L, L)])
                for mb in range(2)]
        iv1 += [jnp.where(uf1, fstv[mb], msgs1[pl.ds(p1 * MSG + mb * L, L)])
                for mb in range(2)]

        for k in range(HID + MSG):
            v0 = jnp.broadcast_to(iv0[k // L][k % L], (L,))
            v1 = jnp.broadcast_to(iv1[k // L][k % L], (L,))
            par = k % 2
            for hb in range(4):
                w = wns[pl.ds(k * HID + hb * L, L)]
                a0[par][hb] = a0[par][hb] + w * v0
                a1[par][hb] = a1[par][hb] + w * v1
            if k >= HID:
                for mb in range(2):
                    w = wnm[pl.ds(k * MSG + mb * L, L)]
                    m0[par][mb] = m0[par][mb] + w * v0
                    m1[par][mb] = m1[par][mb] + w * v1

        ns0 = [jnp.maximum(a0[0][hb] + a0[1][hb], 0.0) for hb in range(4)]
        ns1 = [jnp.maximum(a1[0][hb] + a1[1][hb], 0.0) for hb in range(4)]
        for hb in range(4):
            pred0[pl.ds(nd0 * HID + hb * L, L)] = ns0[hb]
            pred1[pl.ds(nd1 * HID + hb * L, L)] = ns1[hb]

        for k in range(HID):
            n0 = jnp.broadcast_to(ns0[k // L][k % L], (L,))
            n1 = jnp.broadcast_to(ns1[k // L][k % L], (L,))
            par = k % 2
            for mb in range(2):
                w = wnm[pl.ds(k * MSG + mb * L, L)]
                m0[par][mb] = m0[par][mb] + w * n0
                m1[par][mb] = m1[par][mb] + w * n1
        m0 = [m0[0][mb] + m0[1][mb] for mb in range(2)]
        m1 = [m1[0][mb] + m1[1][mb] for mb in range(2)]

        for mb in range(2):
            msgs0[pl.ds(t * MSG + mb * L, L)] = m0[mb]
            msgs1[pl.ds(t * MSG + mb * L, L)] = m1[mb]
        return carry

    lax.fori_loop(0, T, step, 0)

    for hb in range(4):
        fin[pl.ds(hb * L, L)] = pred0[pl.ds(s0 * HID + hb * L, L)]
        fin[pl.ds(HID + hb * L, L)] = pred1[pl.ds((s0 + 1) * HID + hb * L, L)]
    pltpu.sync_copy(fin, out_hbm.at[pl.ds(s0 * HID, SIMS * HID)])


def kernel(x, edge_index, nodes, parents, first_message,
           W_enc, b_enc, W_ns, b_ns, W_nm, b_nm, W_dec, b_dec):
    del edge_index
    f32 = jnp.float32

    enc = pl.pallas_call(
        _encode_body,
        out_shape=jax.ShapeDtypeStruct((N, HID), f32),
    )(x, W_enc.T, b_enc.reshape(1, HID))

    sim = pl.kernel(
        _sim_body,
        out_type=jax.ShapeDtypeStruct((S * HID,), f32),
        mesh=plsc.VectorSubcoreMesh(core_axis_name="c", subcore_axis_name="s"),
        scratch_types=[
            pltpu.VMEM((N * HID,), f32),
            pltpu.VMEM((N * HID,), f32),
            pltpu.VMEM((T * MSG,), f32),
            pltpu.VMEM((T * MSG,), f32),
            pltpu.VMEM((T * L,), jnp.int32),
            pltpu.VMEM((T * L,), jnp.int32),
            pltpu.VMEM(((HID + MSG) * HID,), f32),
            pltpu.VMEM(((HID + MSG) * MSG,), f32),
            pltpu.VMEM((HID,), f32),
            pltpu.VMEM((MSG,), f32),
            pltpu.VMEM((MSG,), f32),
            pltpu.VMEM((SIMS * HID,), f32),
        ],
    )
    sched = jnp.stack([nodes.astype(jnp.int32), parents.astype(jnp.int32)],
                      axis=-1)
    sched = jnp.pad(sched, ((0, 0), (0, 0), (0, L - 2)))
    final = sim(enc.reshape(N * HID),
                sched.reshape(S * T * L),
                first_message.reshape(MSG),
                W_ns.T.reshape((HID + MSG) * HID),
                b_ns, W_nm.T.reshape((HID + MSG) * MSG), b_nm)

    return pl.pallas_call(
        _decode_body,
        out_shape=jax.ShapeDtypeStruct((S, OUT_F), f32),
    )(final.reshape(S, HID), W_dec.T, b_dec.reshape(1, OUT_F))
